# Initial kernel scaffold; baseline (speedup 1.0000x reference)
import jax
import jax.numpy as jnp
from jax.experimental import pallas as pl


def kernel(x, edge_index, batch, W1, b1, W2, b2, W3, b3, Wl, bl):
    raise NotImplementedError("write your pallas kernel here")


# speedup vs baseline: 9.4701x; 9.4701x over previous
import functools

import jax
import jax.numpy as jnp
from jax import lax
from jax.experimental import pallas as pl
from jax.experimental.pallas import tpu as pltpu
from jax.experimental.pallas import tpu_sc as plsc

_N = 50000
_E = 800000
_DIN = 128
_DH = 64
_HF = 32
_NG = 512
_NS = 16
_CW = 125
_RE = _E // _CW
_RET = _RE // _NS
_STN = _N // _NS
_RN = _N // _CW
_RNT = _RN // _NS


def _sc_mesh():
    return plsc.VectorSubcoreMesh(core_axis_name="c", subcore_axis_name="s")


def _prep_body(dst_rs, batch_rs, dinv_out, cnt_out,
               degacc, cntacc, ones_t, idx, stripe_t):
    c = lax.axis_index("c")
    s = lax.axis_index("s")

    def fill_ones(i, carry):
        ones_t[i, :] = jnp.full((16,), 1.0, jnp.float32)
        return carry
    lax.fori_loop(0, _CW, fill_ones, 0)

    @pl.when(c == 0)
    def _():
        def init(i, carry):
            pltpu.sync_copy(ones_t, degacc.at[pl.ds(s * _STN + i * _CW, _CW)])
            return carry
        lax.fori_loop(0, _STN // _CW, init, 0)

    @pl.when(c == 1)
    def _():
        def zf(i, carry):
            stripe_t[i, :] = jnp.zeros((16,), jnp.float32)
            return carry
        lax.fori_loop(0, _NG // _NS, zf, 0)
        pltpu.sync_copy(stripe_t.at[pl.ds(0, _NG // _NS)],
                        cntacc.at[pl.ds(s * (_NG // _NS), _NG // _NS)])

    plsc.subcore_barrier()

    @pl.when(c == 0)
    def _():
        def edge(i, carry):
            pltpu.sync_copy(dst_rs.at[s * _RET + i], idx)
            pltpu.sync_copy(ones_t, degacc.at[idx], add=True)
            return carry
        lax.fori_loop(0, _RET, edge, 0)

    @pl.when(c == 1)
    def _():
        def node(i, carry):
            pltpu.sync_copy(batch_rs.at[s * _RNT + i], idx)
            pltpu.sync_copy(ones_t, cntacc.at[idx], add=True)
            return carry
        lax.fori_loop(0, _RNT, node, 0)

    plsc.subcore_barrier()

    @pl.when(c == 0)
    def _():
        pltpu.sync_copy(degacc.at[pl.ds(s * _STN, _STN)], stripe_t)

        def newton(i, carry):
            d = stripe_t[i, :]
            bits = plsc.bitcast(d, jnp.int32)
            y = plsc.bitcast(jnp.int32(0x5F3759DF) - (bits >> 1), jnp.float32)
            hd = d * 0.5
            y = y * (1.5 - hd * y * y)
            y = y * (1.5 - hd * y * y)
            y = y * (1.5 - hd * y * y)
            stripe_t[i, :] = y
            return carry
        lax.fori_loop(0, _STN, newton, 0)
        pltpu.sync_copy(stripe_t, dinv_out.at[pl.ds(s * _STN, _STN)])

    @pl.when((c == 1) & (s == 0))
    def _():
        pltpu.sync_copy(cntacc, cnt_out)


@jax.jit
def _prep(dst_rs, batch_rs):
    f = pl.kernel(
        _prep_body,
        out_type=(jax.ShapeDtypeStruct((_N, 16), jnp.float32),
                  jax.ShapeDtypeStruct((_NG, 16), jnp.float32)),
        mesh=_sc_mesh(),
        compiler_params=pltpu.CompilerParams(use_tc_tiling_on_sc=False, needs_layout_passes=False),
        scratch_types=[
            pltpu.VMEM_SHARED((_N, 16), jnp.float32),
            pltpu.VMEM_SHARED((_NG, 16), jnp.float32),
            pltpu.VMEM((_CW, 16), jnp.float32),
            pltpu.VMEM((_CW,), jnp.int32),
            pltpu.VMEM((_STN, 16), jnp.float32),
        ],
    )
    return f(dst_rs, batch_rs)


def _agg_body(g2n, src0_rs, src1_rs, dst_rs, zeros, out,
              acc, idx_g, idx_s, rows, sem):
    c = lax.axis_index("c")
    s = lax.axis_index("s")
    pltpu.sync_copy(zeros.at[pl.ds(s * _STN, _STN)],
                    acc.at[pl.ds(s * _STN, _STN)])
    plsc.subcore_barrier()

    def edge_loop(src_rs):
        def body(i, carry):
            r = s * _RET + i
            pltpu.sync_copy(src_rs.at[r], idx_g)
            pltpu.sync_copy(dst_rs.at[r], idx_s)
            pltpu.async_copy(g2n.at[idx_g], rows, sem).wait()
            pltpu.sync_copy(rows, acc.at[idx_s], add=True)
            return carry
        lax.fori_loop(0, _RET, body, 0)

    @pl.when(c == 0)
    def _():
        edge_loop(src0_rs)

    @pl.when(c == 1)
    def _():
        edge_loop(src1_rs)

    plsc.subcore_barrier()
    pltpu.sync_copy(acc.at[pl.ds(s * _STN, _STN)],
                    out.at[c, pl.ds(s * _STN, _STN)])


@jax.jit
def _agg(g2n, src0_rs, src1_rs, dst_rs, zeros):
    f = pl.kernel(
        _agg_body,
        out_type=jax.ShapeDtypeStruct((2, _N, _HF), jnp.float32),
        mesh=_sc_mesh(),
        compiler_params=pltpu.CompilerParams(use_tc_tiling_on_sc=False, needs_layout_passes=False),
        scratch_types=[
            pltpu.VMEM_SHARED((_N, _HF), jnp.float32),
            pltpu.VMEM((_CW,), jnp.int32),
            pltpu.VMEM((_CW,), jnp.int32),
            pltpu.VMEM((_CW, _HF), jnp.float32),
            pltpu.SemaphoreType.DMA,
        ],
    )
    return f(g2n, src0_rs, src1_rs, dst_rs, zeros)


def _pool_body(h2n, batch_rs, zeros, out, acc, idx, rows, sem):
    c = lax.axis_index("c")
    s = lax.axis_index("s")

    @pl.when(s == 0)
    def _():
        pltpu.sync_copy(zeros.at[pl.ds(0, _NG)], acc)
    plsc.subcore_barrier()

    def body(i, carry):
        r = s * _RNT + i
        pltpu.sync_copy(batch_rs.at[r], idx)
        pltpu.sync_copy(h2n.at[pl.ds(c * _N + r * _CW, _CW)], rows)
        pltpu.sync_copy(rows, acc.at[idx], add=True)
        return carry
    lax.fori_loop(0, _RNT, body, 0)

    plsc.subcore_barrier()

    @pl.when(s == 0)
    def _():
        pltpu.sync_copy(acc, out.at[c])


@jax.jit
def _pool(h2n, batch_rs, zeros):
    f = pl.kernel(
        _pool_body,
        out_type=jax.ShapeDtypeStruct((2, _NG, _HF), jnp.float32),
        mesh=_sc_mesh(),
        compiler_params=pltpu.CompilerParams(use_tc_tiling_on_sc=False, needs_layout_passes=False),
        scratch_types=[
            pltpu.VMEM_SHARED((_NG, _HF), jnp.float32),
            pltpu.VMEM((_CW,), jnp.int32),
            pltpu.VMEM((_CW, _HF), jnp.float32),
            pltpu.SemaphoreType.DMA,
        ],
    )
    return f(h2n, batch_rs, zeros)


_BLK = 2000


def _tc1_body(x_ref, w_ref, dinv_ref, out_ref):
    g = jnp.dot(x_ref[...], w_ref[...], preferred_element_type=jnp.float32)
    g = g * dinv_ref[:, :1]
    out_ref[0] = g[:, :_HF]
    out_ref[1] = g[:, _HF:]


@jax.jit
def _tc1(x, W1, dinv_w):
    return pl.pallas_call(
        _tc1_body,
        grid=(_N // _BLK,),
        in_specs=[
            pl.BlockSpec((_BLK, _DIN), lambda i: (i, 0)),
            pl.BlockSpec((_DIN, _DH), lambda i: (0, 0)),
            pl.BlockSpec((_BLK, 16), lambda i: (i, 0)),
        ],
        out_specs=pl.BlockSpec((2, _BLK, _HF), lambda i: (0, i, 0)),
        out_shape=jax.ShapeDtypeStruct((2, _N, _HF), jnp.float32),
    )(x, W1, dinv_w)


def _tcmid_body(s_ref, g_ref, dinv_ref, b_ref, w_ref, out_ref):
    dinv = dinv_ref[:, :1]
    sf = jnp.concatenate([s_ref[0], s_ref[1]], axis=1)
    gf = jnp.concatenate([g_ref[0], g_ref[1]], axis=1)
    xn = jnp.maximum(dinv * (sf + gf) + b_ref[...], 0.0)
    g2 = jnp.dot(xn, w_ref[...], preferred_element_type=jnp.float32) * dinv
    out_ref[0] = g2[:, :_HF]
    out_ref[1] = g2[:, _HF:]


@jax.jit
def _tcmid(s_prev, g_prev, dinv_w, b_prev, W):
    return pl.pallas_call(
        _tcmid_body,
        grid=(_N // _BLK,),
        in_specs=[
            pl.BlockSpec((2, _BLK, _HF), lambda i: (0, i, 0)),
            pl.BlockSpec((2, _BLK, _HF), lambda i: (0, i, 0)),
            pl.BlockSpec((_BLK, 16), lambda i: (i, 0)),
            pl.BlockSpec((1, _DH), lambda i: (0, 0)),
            pl.BlockSpec((_DH, _DH), lambda i: (0, 0)),
        ],
        out_specs=pl.BlockSpec((2, _BLK, _HF), lambda i: (0, i, 0)),
        out_shape=jax.ShapeDtypeStruct((2, _N, _HF), jnp.float32),
    )(s_prev, g_prev, dinv_w, b_prev, W)


def _ep3_body(s_ref, g_ref, dinv_ref, out_ref):
    dinv = dinv_ref[:, :1]
    sf = jnp.concatenate([s_ref[0], s_ref[1]], axis=1)
    gf = jnp.concatenate([g_ref[0], g_ref[1]], axis=1)
    h = dinv * (sf + gf)
    out_ref[0] = h[:, :_HF]
    out_ref[1] = h[:, _HF:]


@jax.jit
def _ep3(s3, g3, dinv_w):
    return pl.pallas_call(
        _ep3_body,
        grid=(_N // _BLK,),
        in_specs=[
            pl.BlockSpec((2, _BLK, _HF), lambda i: (0, i, 0)),
            pl.BlockSpec((2, _BLK, _HF), lambda i: (0, i, 0)),
            pl.BlockSpec((_BLK, 16), lambda i: (i, 0)),
        ],
        out_specs=pl.BlockSpec((2, _BLK, _HF), lambda i: (0, i, 0)),
        out_shape=jax.ShapeDtypeStruct((2, _N, _HF), jnp.float32),
    )(s3, g3, dinv_w)


def _fin_body(p_ref, cnt_ref, b3_ref, wl_ref, bl_ref, out_ref):
    t = jnp.concatenate([p_ref[0], p_ref[1]], axis=1)
    cnt = cnt_ref[:, :1]
    pooled = (t + cnt * b3_ref[...]) / jnp.maximum(cnt, 1.0)
    out_ref[...] = (jnp.dot(pooled, wl_ref[...],
                            preferred_element_type=jnp.float32) + bl_ref[...])


@jax.jit
def _fin(p, cnt_w, b3, Wl, bl):
    return pl.pallas_call(
        _fin_body,
        out_shape=jax.ShapeDtypeStruct((_NG, 8), jnp.float32),
    )(p, cnt_w, b3, Wl, bl)


def kernel(x, edge_index, batch, W1, b1, W2, b2, W3, b3, Wl, bl):
    src = edge_index[0].astype(jnp.int32)
    dst = edge_index[1].astype(jnp.int32)
    src0_rs = src.reshape(_RE, _CW)
    src1_rs = (src + _N).reshape(_RE, _CW)
    dst_rs = dst.reshape(_RE, _CW)
    batch_rs = batch.astype(jnp.int32).reshape(_RN, _CW)
    zeros = jnp.zeros((_N, _HF), jnp.float32)

    dinv_w, cnt_w = _prep(dst_rs, batch_rs)
    g1 = _tc1(x, W1, dinv_w)
    s1 = _agg(g1.reshape(2 * _N, _HF), src0_rs, src1_rs, dst_rs, zeros)
    g2 = _tcmid(s1, g1, dinv_w, b1.reshape(1, _DH), W2)
    s2 = _agg(g2.reshape(2 * _N, _HF), src0_rs, src1_rs, dst_rs, zeros)
    g3 = _tcmid(s2, g2, dinv_w, b2.reshape(1, _DH), W3)
    s3 = _agg(g3.reshape(2 * _N, _HF), src0_rs, src1_rs, dst_rs, zeros)
    h3 = _ep3(s3, g3, dinv_w)
    p = _pool(h3.reshape(2 * _N, _HF), batch_rs, zeros)
    return _fin(p, cnt_w, b3.reshape(1, _DH), Wl, bl.reshape(1, 8))


# speedup vs baseline: 17.6675x; 1.8656x over previous
import functools

import jax
import jax.numpy as jnp
from jax import lax
from jax.experimental import pallas as pl
from jax.experimental.pallas import tpu as pltpu
from jax.experimental.pallas import tpu_sc as plsc

_N = 50000
_E = 800000
_DIN = 128
_DH = 64
_HF = 32
_NG = 512
_NS = 16
_CW = 125
_RE = _E // _CW
_RET = _RE // _NS
_STN = _N // _NS
_RN = _N // _CW
_RNT = _RN // _NS
_K = 4


def _sc_mesh():
    return plsc.VectorSubcoreMesh(core_axis_name="c", subcore_axis_name="s")


def _prep_body(dst_rs, batch_rs, dinv_out, cnt_out,
               degacc, cntacc, ones_t, idx, stripe_t):
    c = lax.axis_index("c")
    s = lax.axis_index("s")

    def fill_ones(i, carry):
        ones_t[i, :] = jnp.full((16,), 1.0, jnp.float32)
        return carry
    lax.fori_loop(0, _CW, fill_ones, 0)

    @pl.when(c == 0)
    def _():
        def init(i, carry):
            pltpu.sync_copy(ones_t, degacc.at[pl.ds(s * _STN + i * _CW, _CW)])
            return carry
        lax.fori_loop(0, _STN // _CW, init, 0)

    @pl.when(c == 1)
    def _():
        def zf(i, carry):
            stripe_t[i, :] = jnp.zeros((16,), jnp.float32)
            return carry
        lax.fori_loop(0, _NG // _NS, zf, 0)
        pltpu.sync_copy(stripe_t.at[pl.ds(0, _NG // _NS)],
                        cntacc.at[pl.ds(s * (_NG // _NS), _NG // _NS)])

    plsc.subcore_barrier()

    @pl.when(c == 0)
    def _():
        def edge(i, carry):
            pltpu.sync_copy(dst_rs.at[s * _RET + i], idx)
            pltpu.sync_copy(ones_t, degacc.at[idx], add=True)
            return carry
        lax.fori_loop(0, _RET, edge, 0)

    @pl.when(c == 1)
    def _():
        def node(i, carry):
            pltpu.sync_copy(batch_rs.at[s * _RNT + i], idx)
            pltpu.sync_copy(ones_t, cntacc.at[idx], add=True)
            return carry
        lax.fori_loop(0, _RNT, node, 0)

    plsc.subcore_barrier()

    @pl.when(c == 0)
    def _():
        pltpu.sync_copy(degacc.at[pl.ds(s * _STN, _STN)], stripe_t)

        def newton(i, carry):
            d = stripe_t[i, :]
            bits = plsc.bitcast(d, jnp.int32)
            y = plsc.bitcast(jnp.int32(0x5F3759DF) - (bits >> 1), jnp.float32)
            hd = d * 0.5
            y = y * (1.5 - hd * y * y)
            y = y * (1.5 - hd * y * y)
            y = y * (1.5 - hd * y * y)
            stripe_t[i, :] = y
            return carry
        lax.fori_loop(0, _STN, newton, 0)
        pltpu.sync_copy(stripe_t, dinv_out.at[pl.ds(s * _STN, _STN)])

    @pl.when((c == 1) & (s == 0))
    def _():
        pltpu.sync_copy(cntacc, cnt_out)


@jax.jit
def _prep(dst_rs, batch_rs):
    f = pl.kernel(
        _prep_body,
        out_type=(jax.ShapeDtypeStruct((_N, 16), jnp.float32),
                  jax.ShapeDtypeStruct((_NG, 16), jnp.float32)),
        mesh=_sc_mesh(),
        compiler_params=pltpu.CompilerParams(use_tc_tiling_on_sc=False, needs_layout_passes=False),
        scratch_types=[
            pltpu.VMEM_SHARED((_N, 16), jnp.float32),
            pltpu.VMEM_SHARED((_NG, 16), jnp.float32),
            pltpu.VMEM((_CW, 16), jnp.float32),
            pltpu.VMEM((_CW,), jnp.int32),
            pltpu.VMEM((_STN, 16), jnp.float32),
        ],
    )
    return f(dst_rs, batch_rs)


def _agg_body(g2n, src0_rs, src1_rs, dst_rs, zeros, out,
              acc, idx_g, idx_s, rows, gsem, ssem):
    c = lax.axis_index("c")
    s = lax.axis_index("s")
    pltpu.sync_copy(zeros.at[pl.ds(s * _STN, _STN)],
                    acc.at[pl.ds(s * _STN, _STN)])
    plsc.subcore_barrier()

    def edge_loop(src_rs):
        def chunk(cc, carry):
            r0 = s * _RET + cc * _K
            pltpu.sync_copy(src_rs.at[pl.ds(r0, _K)], idx_g)
            pltpu.sync_copy(dst_rs.at[pl.ds(r0, _K)], idx_s)
            for j in range(_K):
                pltpu.async_copy(g2n.at[idx_g.at[j]], rows.at[j], gsem)
            for j in range(_K):
                pltpu.make_async_copy(g2n.at[idx_g.at[j]], rows.at[j],
                                      gsem).wait()
                pltpu.async_copy(rows.at[j], acc.at[idx_s.at[j]], ssem,
                                 add=True)
            for j in range(_K):
                pltpu.make_async_copy(rows.at[j], acc.at[idx_s.at[j]],
                                      ssem).wait()
            return carry
        lax.fori_loop(0, _RET // _K, chunk, 0)

    @pl.when(c == 0)
    def _():
        edge_loop(src0_rs)

    @pl.when(c == 1)
    def _():
        edge_loop(src1_rs)

    plsc.subcore_barrier()
    pltpu.sync_copy(acc.at[pl.ds(s * _STN, _STN)],
                    out.at[c, pl.ds(s * _STN, _STN)])


@jax.jit
def _agg(g2n, src0_rs, src1_rs, dst_rs, zeros):
    f = pl.kernel(
        _agg_body,
        out_type=jax.ShapeDtypeStruct((2, _N, _HF), jnp.float32),
        mesh=_sc_mesh(),
        compiler_params=pltpu.CompilerParams(use_tc_tiling_on_sc=False, needs_layout_passes=False),
        scratch_types=[
            pltpu.VMEM_SHARED((_N, _HF), jnp.float32),
            pltpu.VMEM((_K, _CW), jnp.int32),
            pltpu.VMEM((_K, _CW), jnp.int32),
            pltpu.VMEM((_K, _CW, _HF), jnp.float32),
            pltpu.SemaphoreType.DMA,
            pltpu.SemaphoreType.DMA,
        ],
    )
    return f(g2n, src0_rs, src1_rs, dst_rs, zeros)


def _pool_body(h2n, batch_rs, zeros, out, acc, idx, rows, sem):
    c = lax.axis_index("c")
    s = lax.axis_index("s")

    @pl.when(s == 0)
    def _():
        pltpu.sync_copy(zeros.at[pl.ds(0, _NG)], acc)
    plsc.subcore_barrier()

    def body(i, carry):
        r = s * _RNT + i
        pltpu.sync_copy(batch_rs.at[r], idx)
        pltpu.sync_copy(h2n.at[pl.ds(c * _N + r * _CW, _CW)], rows)
        pltpu.sync_copy(rows, acc.at[idx], add=True)
        return carry
    lax.fori_loop(0, _RNT, body, 0)

    plsc.subcore_barrier()

    @pl.when(s == 0)
    def _():
        pltpu.sync_copy(acc, out.at[c])


@jax.jit
def _pool(h2n, batch_rs, zeros):
    f = pl.kernel(
        _pool_body,
        out_type=jax.ShapeDtypeStruct((2, _NG, _HF), jnp.float32),
        mesh=_sc_mesh(),
        compiler_params=pltpu.CompilerParams(use_tc_tiling_on_sc=False, needs_layout_passes=False),
        scratch_types=[
            pltpu.VMEM_SHARED((_NG, _HF), jnp.float32),
            pltpu.VMEM((_CW,), jnp.int32),
            pltpu.VMEM((_CW, _HF), jnp.float32),
            pltpu.SemaphoreType.DMA,
        ],
    )
    return f(h2n, batch_rs, zeros)


_BLK = 2000


def _tc1_body(x_ref, w_ref, dinv_ref, out_ref):
    g = jnp.dot(x_ref[...], w_ref[...], preferred_element_type=jnp.float32)
    g = g * dinv_ref[:, :1]
    out_ref[0] = g[:, :_HF]
    out_ref[1] = g[:, _HF:]


@jax.jit
def _tc1(x, W1, dinv_w):
    return pl.pallas_call(
        _tc1_body,
        grid=(_N // _BLK,),
        in_specs=[
            pl.BlockSpec((_BLK, _DIN), lambda i: (i, 0)),
            pl.BlockSpec((_DIN, _DH), lambda i: (0, 0)),
            pl.BlockSpec((_BLK, 16), lambda i: (i, 0)),
        ],
        out_specs=pl.BlockSpec((2, _BLK, _HF), lambda i: (0, i, 0)),
        out_shape=jax.ShapeDtypeStruct((2, _N, _HF), jnp.float32),
    )(x, W1, dinv_w)


def _tcmid_body(s_ref, g_ref, dinv_ref, b_ref, w_ref, out_ref):
    dinv = dinv_ref[:, :1]
    sf = jnp.concatenate([s_ref[0], s_ref[1]], axis=1)
    gf = jnp.concatenate([g_ref[0], g_ref[1]], axis=1)
    xn = jnp.maximum(dinv * (sf + gf) + b_ref[...], 0.0)
    g2 = jnp.dot(xn, w_ref[...], preferred_element_type=jnp.float32) * dinv
    out_ref[0] = g2[:, :_HF]
    out_ref[1] = g2[:, _HF:]


@jax.jit
def _tcmid(s_prev, g_prev, dinv_w, b_prev, W):
    return pl.pallas_call(
        _tcmid_body,
        grid=(_N // _BLK,),
        in_specs=[
            pl.BlockSpec((2, _BLK, _HF), lambda i: (0, i, 0)),
            pl.BlockSpec((2, _BLK, _HF), lambda i: (0, i, 0)),
            pl.BlockSpec((_BLK, 16), lambda i: (i, 0)),
            pl.BlockSpec((1, _DH), lambda i: (0, 0)),
            pl.BlockSpec((_DH, _DH), lambda i: (0, 0)),
        ],
        out_specs=pl.BlockSpec((2, _BLK, _HF), lambda i: (0, i, 0)),
        out_shape=jax.ShapeDtypeStruct((2, _N, _HF), jnp.float32),
    )(s_prev, g_prev, dinv_w, b_prev, W)


def _ep3_body(s_ref, g_ref, dinv_ref, out_ref):
    dinv = dinv_ref[:, :1]
    sf = jnp.concatenate([s_ref[0], s_ref[1]], axis=1)
    gf = jnp.concatenate([g_ref[0], g_ref[1]], axis=1)
    h = dinv * (sf + gf)
    out_ref[0] = h[:, :_HF]
    out_ref[1] = h[:, _HF:]


@jax.jit
def _ep3(s3, g3, dinv_w):
    return pl.pallas_call(
        _ep3_body,
        grid=(_N // _BLK,),
        in_specs=[
            pl.BlockSpec((2, _BLK, _HF), lambda i: (0, i, 0)),
            pl.BlockSpec((2, _BLK, _HF), lambda i: (0, i, 0)),
            pl.BlockSpec((_BLK, 16), lambda i: (i, 0)),
        ],
        out_specs=pl.BlockSpec((2, _BLK, _HF), lambda i: (0, i, 0)),
        out_shape=jax.ShapeDtypeStruct((2, _N, _HF), jnp.float32),
    )(s3, g3, dinv_w)


def _fin_body(p_ref, cnt_ref, b3_ref, wl_ref, bl_ref, out_ref):
    t = jnp.concatenate([p_ref[0], p_ref[1]], axis=1)
    cnt = cnt_ref[:, :1]
    pooled = (t + cnt * b3_ref[...]) / jnp.maximum(cnt, 1.0)
    out_ref[...] = (jnp.dot(pooled, wl_ref[...],
                            preferred_element_type=jnp.float32) + bl_ref[...])


@jax.jit
def _fin(p, cnt_w, b3, Wl, bl):
    return pl.pallas_call(
        _fin_body,
        out_shape=jax.ShapeDtypeStruct((_NG, 8), jnp.float32),
    )(p, cnt_w, b3, Wl, bl)


def kernel(x, edge_index, batch, W1, b1, W2, b2, W3, b3, Wl, bl):
    src = edge_index[0].astype(jnp.int32)
    dst = edge_index[1].astype(jnp.int32)
    src0_rs = src.reshape(_RE, _CW)
    src1_rs = (src + _N).reshape(_RE, _CW)
    dst_rs = dst.reshape(_RE, _CW)
    batch_rs = batch.astype(jnp.int32).reshape(_RN, _CW)
    zeros = jnp.zeros((_N, _HF), jnp.float32)

    dinv_w, cnt_w = _prep(dst_rs, batch_rs)
    g1 = _tc1(x, W1, dinv_w)
    s1 = _agg(g1.reshape(2 * _N, _HF), src0_rs, src1_rs, dst_rs, zeros)
    g2 = _tcmid(s1, g1, dinv_w, b1.reshape(1, _DH), W2)
    s2 = _agg(g2.reshape(2 * _N, _HF), src0_rs, src1_rs, dst_rs, zeros)
    g3 = _tcmid(s2, g2, dinv_w, b2.reshape(1, _DH), W3)
    s3 = _agg(g3.reshape(2 * _N, _HF), src0_rs, src1_rs, dst_rs, zeros)
    h3 = _ep3(s3, g3, dinv_w)
    p = _pool(h3.reshape(2 * _N, _HF), batch_rs, zeros)
    return _fin(p, cnt_w, b3.reshape(1, _DH), Wl, bl.reshape(1, 8))


# speedup vs baseline: 20.2661x; 1.1471x over previous
import functools

import jax
import jax.numpy as jnp
from jax import lax
from jax.experimental import pallas as pl
from jax.experimental.pallas import tpu as pltpu
from jax.experimental.pallas import tpu_sc as plsc

_N = 50000
_E = 800000
_DIN = 128
_DH = 64
_HF = 32
_NG = 512
_NS = 16
_CW = 125
_RE = _E // _CW
_RET = _RE // _NS
_STN = _N // _NS
_RN = _N // _CW
_RNT = _RN // _NS
_K = 4


def _sc_mesh():
    return plsc.VectorSubcoreMesh(core_axis_name="c", subcore_axis_name="s")


def _prep_body(dst_rs, dinv_out, degacc, ones_t, idx, stripe_t, ssem):
    c = lax.axis_index("c")
    s = lax.axis_index("s")

    def fill_ones(i, carry):
        ones_t[i, :] = jnp.full((16,), 1.0, jnp.float32)
        return carry
    lax.fori_loop(0, _CW, fill_ones, 0)

    @pl.when(c == 0)
    def _():
        def init(i, carry):
            pltpu.sync_copy(ones_t, degacc.at[pl.ds(s * _STN + i * _CW, _CW)])
            return carry
        lax.fori_loop(0, _STN // _CW, init, 0)

    plsc.subcore_barrier()

    @pl.when(c == 0)
    def _():
        def chunk(cc, carry):
            r0 = s * _RET + cc * _K
            pltpu.sync_copy(dst_rs.at[pl.ds(r0, _K)], idx)
            for j in range(_K):
                pltpu.async_copy(ones_t, degacc.at[idx.at[j]], ssem,
                                 add=True)
            for j in range(_K):
                pltpu.make_async_copy(ones_t, degacc.at[idx.at[j]],
                                      ssem).wait()
            return carry
        lax.fori_loop(0, _RET // _K, chunk, 0)

    plsc.subcore_barrier()

    @pl.when(c == 0)
    def _():
        pltpu.sync_copy(degacc.at[pl.ds(s * _STN, _STN)], stripe_t)

        def newton(i, carry):
            d = stripe_t[i, :]
            bits = plsc.bitcast(d, jnp.int32)
            y = plsc.bitcast(jnp.int32(0x5F3759DF) - (bits >> 1), jnp.float32)
            hd = d * 0.5
            y = y * (1.5 - hd * y * y)
            y = y * (1.5 - hd * y * y)
            y = y * (1.5 - hd * y * y)
            stripe_t[i, :] = y
            return carry
        lax.fori_loop(0, _STN, newton, 0)
        pltpu.sync_copy(stripe_t, dinv_out.at[pl.ds(s * _STN, _STN)])


@jax.jit
def _prep(dst_rs):
    f = pl.kernel(
        _prep_body,
        out_type=jax.ShapeDtypeStruct((_N, 16), jnp.float32),
        mesh=_sc_mesh(),
        compiler_params=pltpu.CompilerParams(use_tc_tiling_on_sc=False, needs_layout_passes=False),
        scratch_types=[
            pltpu.VMEM_SHARED((_N, 16), jnp.float32),
            pltpu.VMEM((_CW, 16), jnp.float32),
            pltpu.VMEM((_K, _CW), jnp.int32),
            pltpu.VMEM((_STN, 16), jnp.float32),
            pltpu.SemaphoreType.DMA,
        ],
    )
    return f(dst_rs)


def _agg_body(g2n, src0_rs, src1_rs, dst_rs, zeros, out,
              acc, idx_g, idx_s, rows, gsem, ssem):
    c = lax.axis_index("c")
    s = lax.axis_index("s")
    pltpu.sync_copy(zeros.at[pl.ds(s * _STN, _STN)],
                    acc.at[pl.ds(s * _STN, _STN)])
    plsc.subcore_barrier()

    def edge_loop(src_rs):
        def chunk(cc, carry):
            r0 = s * _RET + cc * _K
            pltpu.sync_copy(src_rs.at[pl.ds(r0, _K)], idx_g)
            pltpu.sync_copy(dst_rs.at[pl.ds(r0, _K)], idx_s)
            for j in range(_K):
                pltpu.async_copy(g2n.at[idx_g.at[j]], rows.at[j], gsem)
            for j in range(_K):
                pltpu.make_async_copy(g2n.at[idx_g.at[j]], rows.at[j],
                                      gsem).wait()
                pltpu.async_copy(rows.at[j], acc.at[idx_s.at[j]], ssem,
                                 add=True)
            for j in range(_K):
                pltpu.make_async_copy(rows.at[j], acc.at[idx_s.at[j]],
                                      ssem).wait()
            return carry
        lax.fori_loop(0, _RET // _K, chunk, 0)

    @pl.when(c == 0)
    def _():
        edge_loop(src0_rs)

    @pl.when(c == 1)
    def _():
        edge_loop(src1_rs)

    plsc.subcore_barrier()
    pltpu.sync_copy(acc.at[pl.ds(s * _STN, _STN)],
                    out.at[c, pl.ds(s * _STN, _STN)])


@jax.jit
def _agg(g2n, src0_rs, src1_rs, dst_rs, zeros):
    f = pl.kernel(
        _agg_body,
        out_type=jax.ShapeDtypeStruct((2, _N, _HF), jnp.float32),
        mesh=_sc_mesh(),
        compiler_params=pltpu.CompilerParams(use_tc_tiling_on_sc=False, needs_layout_passes=False),
        scratch_types=[
            pltpu.VMEM_SHARED((_N, _HF), jnp.float32),
            pltpu.VMEM((_K, _CW), jnp.int32),
            pltpu.VMEM((_K, _CW), jnp.int32),
            pltpu.VMEM((_K, _CW, _HF), jnp.float32),
            pltpu.SemaphoreType.DMA,
            pltpu.SemaphoreType.DMA,
        ],
    )
    return f(g2n, src0_rs, src1_rs, dst_rs, zeros)


_BLK = 2000


def _tc1_body(x_ref, w_ref, dinv_ref, out_ref):
    g = jnp.dot(x_ref[...], w_ref[...], preferred_element_type=jnp.float32)
    g = g * dinv_ref[:, :1]
    out_ref[0] = g[:, :_HF]
    out_ref[1] = g[:, _HF:]


@jax.jit
def _tc1(x, W1, dinv_w):
    return pl.pallas_call(
        _tc1_body,
        grid=(_N // _BLK,),
        in_specs=[
            pl.BlockSpec((_BLK, _DIN), lambda i: (i, 0)),
            pl.BlockSpec((_DIN, _DH), lambda i: (0, 0)),
            pl.BlockSpec((_BLK, 16), lambda i: (i, 0)),
        ],
        out_specs=pl.BlockSpec((2, _BLK, _HF), lambda i: (0, i, 0)),
        out_shape=jax.ShapeDtypeStruct((2, _N, _HF), jnp.float32),
    )(x, W1, dinv_w)


def _tcmid_body(s_ref, g_ref, dinv_ref, b_ref, w_ref, out_ref):
    dinv = dinv_ref[:, :1]
    sf = jnp.concatenate([s_ref[0], s_ref[1]], axis=1)
    gf = jnp.concatenate([g_ref[0], g_ref[1]], axis=1)
    xn = jnp.maximum(dinv * (sf + gf) + b_ref[...], 0.0)
    g2 = jnp.dot(xn, w_ref[...], preferred_element_type=jnp.float32) * dinv
    out_ref[0] = g2[:, :_HF]
    out_ref[1] = g2[:, _HF:]


@jax.jit
def _tcmid(s_prev, g_prev, dinv_w, b_prev, W):
    return pl.pallas_call(
        _tcmid_body,
        grid=(_N // _BLK,),
        in_specs=[
            pl.BlockSpec((2, _BLK, _HF), lambda i: (0, i, 0)),
            pl.BlockSpec((2, _BLK, _HF), lambda i: (0, i, 0)),
            pl.BlockSpec((_BLK, 16), lambda i: (i, 0)),
            pl.BlockSpec((1, _DH), lambda i: (0, 0)),
            pl.BlockSpec((_DH, _DH), lambda i: (0, 0)),
        ],
        out_specs=pl.BlockSpec((2, _BLK, _HF), lambda i: (0, i, 0)),
        out_shape=jax.ShapeDtypeStruct((2, _N, _HF), jnp.float32),
    )(s_prev, g_prev, dinv_w, b_prev, W)


def _poolmm_body(s_ref, g_ref, dinv_ref, batch_ref, sum_ref, cnt_ref):
    i = pl.program_id(0)
    dinv = dinv_ref[:, :1]
    sf = jnp.concatenate([s_ref[0], s_ref[1]], axis=1)
    gf = jnp.concatenate([g_ref[0], g_ref[1]], axis=1)
    h = dinv * (sf + gf)
    ids = batch_ref[:, :1]
    gidx = lax.broadcasted_iota(jnp.int32, (_BLK, _NG), 1)
    oh = (ids == gidx).astype(jnp.float32)
    psum = lax.dot_general(oh, h, (((0,), (0,)), ((), ())),
                           preferred_element_type=jnp.float32)
    pcnt = jnp.sum(oh, axis=0)[None, :]

    @pl.when(i == 0)
    def _():
        sum_ref[...] = jnp.zeros_like(sum_ref)
        cnt_ref[...] = jnp.zeros_like(cnt_ref)
    sum_ref[...] += psum
    cnt_ref[...] += pcnt


@jax.jit
def _poolmm(s3, g3, dinv_w, batch_col):
    return pl.pallas_call(
        _poolmm_body,
        grid=(_N // _BLK,),
        in_specs=[
            pl.BlockSpec((2, _BLK, _HF), lambda i: (0, i, 0)),
            pl.BlockSpec((2, _BLK, _HF), lambda i: (0, i, 0)),
            pl.BlockSpec((_BLK, 16), lambda i: (i, 0)),
            pl.BlockSpec((_BLK, 1), lambda i: (i, 0)),
        ],
        out_specs=[
            pl.BlockSpec((_NG, _DH), lambda i: (0, 0)),
            pl.BlockSpec((1, _NG), lambda i: (0, 0)),
        ],
        out_shape=[jax.ShapeDtypeStruct((_NG, _DH), jnp.float32),
                   jax.ShapeDtypeStruct((1, _NG), jnp.float32)],
    )(s3, g3, dinv_w, batch_col)


def _fin_body(t_ref, cnt_ref, b3_ref, wl_ref, bl_ref, out_ref):
    t = t_ref[...]
    cnt = jnp.reshape(cnt_ref[0], (_NG, 1))
    pooled = (t + cnt * b3_ref[...]) / jnp.maximum(cnt, 1.0)
    out_ref[...] = (jnp.dot(pooled, wl_ref[...],
                            preferred_element_type=jnp.float32) + bl_ref[...])


@jax.jit
def _fin(t, cnt2, b3, Wl, bl):
    return pl.pallas_call(
        _fin_body,
        out_shape=jax.ShapeDtypeStruct((_NG, 8), jnp.float32),
    )(t, cnt2, b3, Wl, bl)


def kernel(x, edge_index, batch, W1, b1, W2, b2, W3, b3, Wl, bl):
    src = edge_index[0].astype(jnp.int32)
    dst = edge_index[1].astype(jnp.int32)
    src0_rs = src.reshape(_RE, _CW)
    src1_rs = (src + _N).reshape(_RE, _CW)
    dst_rs = dst.reshape(_RE, _CW)
    batch_col = batch.astype(jnp.int32).reshape(_N, 1)
    zeros = jnp.zeros((_N, _HF), jnp.float32)

    dinv_w = _prep(dst_rs)
    g1 = _tc1(x, W1, dinv_w)
    s1 = _agg(g1.reshape(2 * _N, _HF), src0_rs, src1_rs, dst_rs, zeros)
    g2 = _tcmid(s1, g1, dinv_w, b1.reshape(1, _DH), W2)
    s2 = _agg(g2.reshape(2 * _N, _HF), src0_rs, src1_rs, dst_rs, zeros)
    g3 = _tcmid(s2, g2, dinv_w, b2.reshape(1, _DH), W3)
    s3 = _agg(g3.reshape(2 * _N, _HF), src0_rs, src1_rs, dst_rs, zeros)
    t, cnt2 = _poolmm(s3, g3, dinv_w, batch_col)
    return _fin(t, cnt2, b3.reshape(1, _DH), Wl, bl.reshape(1, 8))


# speedup vs baseline: 25.9145x; 1.2787x over previous
import functools

import jax
import jax.numpy as jnp
from jax import lax
from jax.experimental import pallas as pl
from jax.experimental.pallas import tpu as pltpu
from jax.experimental.pallas import tpu_sc as plsc

_N = 50000
_E = 800000
_DIN = 128
_DH = 64
_HF = 32
_NG = 512
_NS = 16
_CW = 125
_RE = _E // _CW
_RET = _RE // _NS
_STN = _N // _NS
_RN = _N // _CW
_RNT = _RN // _NS
_K = 4
_KP = 8


def _sc_mesh():
    return plsc.VectorSubcoreMesh(core_axis_name="c", subcore_axis_name="s")


def _prep_body(dst_rs, dinv_out, degacc, ones_t, idx, stripe_t, ssem):
    c = lax.axis_index("c")
    s = lax.axis_index("s")

    def fill_ones(i, carry):
        ones_t[i, :] = jnp.full((16,), 1.0, jnp.float32)
        return carry
    lax.fori_loop(0, _CW, fill_ones, 0)

    @pl.when(c == 0)
    def _():
        def init(i, carry):
            pltpu.sync_copy(ones_t, degacc.at[pl.ds(s * _STN + i * _CW, _CW)])
            return carry
        lax.fori_loop(0, _STN // _CW, init, 0)

    plsc.subcore_barrier()

    @pl.when(c == 0)
    def _():
        def chunk(cc, carry):
            r0 = s * _RET + cc * _KP
            pltpu.sync_copy(dst_rs.at[pl.ds(r0, _KP)], idx)
            for j in range(_KP):
                pltpu.async_copy(ones_t, degacc.at[idx.at[j]], ssem,
                                 add=True)
            for j in range(_KP):
                pltpu.make_async_copy(ones_t, degacc.at[idx.at[j]],
                                      ssem).wait()
            return carry
        lax.fori_loop(0, _RET // _KP, chunk, 0)

    plsc.subcore_barrier()

    @pl.when(c == 0)
    def _():
        pltpu.sync_copy(degacc.at[pl.ds(s * _STN, _STN)], stripe_t)

        def newton(i, carry):
            d = stripe_t[i, :]
            bits = plsc.bitcast(d, jnp.int32)
            y = plsc.bitcast(jnp.int32(0x5F3759DF) - (bits >> 1), jnp.float32)
            hd = d * 0.5
            y = y * (1.5 - hd * y * y)
            y = y * (1.5 - hd * y * y)
            y = y * (1.5 - hd * y * y)
            stripe_t[i, :] = y
            return carry
        lax.fori_loop(0, _STN, newton, 0)
        pltpu.sync_copy(stripe_t, dinv_out.at[pl.ds(s * _STN, _STN)])


@jax.jit
def _prep(dst_rs):
    f = pl.kernel(
        _prep_body,
        out_type=jax.ShapeDtypeStruct((_N, 16), jnp.float32),
        mesh=_sc_mesh(),
        compiler_params=pltpu.CompilerParams(use_tc_tiling_on_sc=False, needs_layout_passes=False),
        scratch_types=[
            pltpu.VMEM_SHARED((_N, 16), jnp.float32),
            pltpu.VMEM((_CW, 16), jnp.float32),
            pltpu.VMEM((_KP, _CW), jnp.int32),
            pltpu.VMEM((_STN, 16), jnp.float32),
            pltpu.SemaphoreType.DMA,
        ],
    )
    return f(dst_rs)


def _agg_body(g2n, src0_rs, src1_rs, dst_rs, zeros, out,
              acc, idx_g, idx_s, rows, gsem, ssem, isem):
    c = lax.axis_index("c")
    s = lax.axis_index("s")
    pltpu.sync_copy(zeros.at[pl.ds(s * _STN, _STN)],
                    acc.at[pl.ds(s * _STN, _STN)])
    plsc.subcore_barrier()

    def edge_loop(src_rs):
        base = s * _RET
        nchunk = _RET // _K
        pltpu.async_copy(src_rs.at[pl.ds(base, _K)], idx_g.at[0], isem)
        pltpu.async_copy(dst_rs.at[pl.ds(base, _K)], idx_s.at[0], isem)

        def chunk(cc, carry):
            cur = lax.rem(cc, 2)
            nxt = lax.rem(cc + 1, 2)
            pltpu.make_async_copy(src_rs.at[pl.ds(base, _K)],
                                  idx_g.at[cur], isem).wait()
            pltpu.make_async_copy(dst_rs.at[pl.ds(base, _K)],
                                  idx_s.at[cur], isem).wait()

            @pl.when(cc + 1 < nchunk)
            def _():
                r1 = base + (cc + 1) * _K
                pltpu.async_copy(src_rs.at[pl.ds(r1, _K)], idx_g.at[nxt],
                                 isem)
                pltpu.async_copy(dst_rs.at[pl.ds(r1, _K)], idx_s.at[nxt],
                                 isem)
            for j in range(_K):
                pltpu.async_copy(g2n.at[idx_g.at[cur, j]], rows.at[j], gsem)
            for j in range(_K):
                pltpu.make_async_copy(g2n.at[idx_g.at[cur, j]], rows.at[j],
                                      gsem).wait()
                pltpu.async_copy(rows.at[j], acc.at[idx_s.at[cur, j]], ssem,
                                 add=True)
            for j in range(_K):
                pltpu.make_async_copy(rows.at[j], acc.at[idx_s.at[cur, j]],
                                      ssem).wait()
            return carry
        lax.fori_loop(0, nchunk, chunk, 0)

    @pl.when(c == 0)
    def _():
        edge_loop(src0_rs)

    @pl.when(c == 1)
    def _():
        edge_loop(src1_rs)

    plsc.subcore_barrier()
    pltpu.sync_copy(acc.at[pl.ds(s * _STN, _STN)],
                    out.at[c, pl.ds(s * _STN, _STN)])


@jax.jit
def _agg(g2n, src0_rs, src1_rs, dst_rs, zeros):
    f = pl.kernel(
        _agg_body,
        out_type=jax.ShapeDtypeStruct((2, _N, _HF), jnp.float32),
        mesh=_sc_mesh(),
        compiler_params=pltpu.CompilerParams(use_tc_tiling_on_sc=False, needs_layout_passes=False),
        scratch_types=[
            pltpu.VMEM_SHARED((_N, _HF), jnp.float32),
            pltpu.VMEM((2, _K, _CW), jnp.int32),
            pltpu.VMEM((2, _K, _CW), jnp.int32),
            pltpu.VMEM((_K, _CW, _HF), jnp.float32),
            pltpu.SemaphoreType.DMA,
            pltpu.SemaphoreType.DMA,
            pltpu.SemaphoreType.DMA,
        ],
    )
    return f(g2n, src0_rs, src1_rs, dst_rs, zeros)


_BLK = 2000


def _tc1_body(x_ref, w_ref, dinv_ref, out_ref):
    g = jnp.dot(x_ref[...], w_ref[...], preferred_element_type=jnp.float32)
    g = g * dinv_ref[:, :1]
    out_ref[0] = g[:, :_HF]
    out_ref[1] = g[:, _HF:]


@jax.jit
def _tc1(x, W1, dinv_w):
    return pl.pallas_call(
        _tc1_body,
        grid=(_N // _BLK,),
        in_specs=[
            pl.BlockSpec((_BLK, _DIN), lambda i: (i, 0)),
            pl.BlockSpec((_DIN, _DH), lambda i: (0, 0)),
            pl.BlockSpec((_BLK, 16), lambda i: (i, 0)),
        ],
        out_specs=pl.BlockSpec((2, _BLK, _HF), lambda i: (0, i, 0)),
        out_shape=jax.ShapeDtypeStruct((2, _N, _HF), jnp.float32),
    )(x, W1, dinv_w)


def _tcmid_body(s_ref, g_ref, dinv_ref, b_ref, w_ref, out_ref):
    dinv = dinv_ref[:, :1]
    sf = jnp.concatenate([s_ref[0], s_ref[1]], axis=1)
    gf = jnp.concatenate([g_ref[0], g_ref[1]], axis=1)
    xn = jnp.maximum(dinv * (sf + gf) + b_ref[...], 0.0)
    g2 = jnp.dot(xn, w_ref[...], preferred_element_type=jnp.float32) * dinv
    out_ref[0] = g2[:, :_HF]
    out_ref[1] = g2[:, _HF:]


@jax.jit
def _tcmid(s_prev, g_prev, dinv_w, b_prev, W):
    return pl.pallas_call(
        _tcmid_body,
        grid=(_N // _BLK,),
        in_specs=[
            pl.BlockSpec((2, _BLK, _HF), lambda i: (0, i, 0)),
            pl.BlockSpec((2, _BLK, _HF), lambda i: (0, i, 0)),
            pl.BlockSpec((_BLK, 16), lambda i: (i, 0)),
            pl.BlockSpec((1, _DH), lambda i: (0, 0)),
            pl.BlockSpec((_DH, _DH), lambda i: (0, 0)),
        ],
        out_specs=pl.BlockSpec((2, _BLK, _HF), lambda i: (0, i, 0)),
        out_shape=jax.ShapeDtypeStruct((2, _N, _HF), jnp.float32),
    )(s_prev, g_prev, dinv_w, b_prev, W)


def _poolmm_body(s_ref, g_ref, dinv_ref, batch_ref, b3_ref, wl_ref, bl_ref,
                 out_ref, sum_ref, cnt_ref):
    i = pl.program_id(0)
    dinv = dinv_ref[:, :1]
    sf = jnp.concatenate([s_ref[0], s_ref[1]], axis=1)
    gf = jnp.concatenate([g_ref[0], g_ref[1]], axis=1)
    h = dinv * (sf + gf)
    ids = batch_ref[:, :1]
    gidx = lax.broadcasted_iota(jnp.int32, (_BLK, _NG), 1)
    oh = (ids == gidx).astype(jnp.float32)
    psum = lax.dot_general(oh, h, (((0,), (0,)), ((), ())),
                           preferred_element_type=jnp.float32)
    pcnt = jnp.sum(oh, axis=0)[None, :]

    @pl.when(i == 0)
    def _():
        sum_ref[...] = jnp.zeros_like(sum_ref)
        cnt_ref[...] = jnp.zeros_like(cnt_ref)
    sum_ref[...] += psum
    cnt_ref[...] += pcnt

    @pl.when(i == _N // _BLK - 1)
    def _():
        t = sum_ref[...]
        cnt = jnp.reshape(cnt_ref[0], (_NG, 1))
        pooled = (t + cnt * b3_ref[...]) / jnp.maximum(cnt, 1.0)
        out_ref[...] = (jnp.dot(pooled, wl_ref[...],
                                preferred_element_type=jnp.float32)
                        + bl_ref[...])


@jax.jit
def _poolmm(s3, g3, dinv_w, batch_col, b3, Wl, bl):
    return pl.pallas_call(
        _poolmm_body,
        grid=(_N // _BLK,),
        in_specs=[
            pl.BlockSpec((2, _BLK, _HF), lambda i: (0, i, 0)),
            pl.BlockSpec((2, _BLK, _HF), lambda i: (0, i, 0)),
            pl.BlockSpec((_BLK, 16), lambda i: (i, 0)),
            pl.BlockSpec((_BLK, 1), lambda i: (i, 0)),
            pl.BlockSpec((1, _DH), lambda i: (0, 0)),
            pl.BlockSpec((_DH, 8), lambda i: (0, 0)),
            pl.BlockSpec((1, 8), lambda i: (0, 0)),
        ],
        out_specs=pl.BlockSpec((_NG, 8), lambda i: (0, 0)),
        out_shape=jax.ShapeDtypeStruct((_NG, 8), jnp.float32),
        scratch_shapes=[
            pltpu.VMEM((_NG, _DH), jnp.float32),
            pltpu.VMEM((1, _NG), jnp.float32),
        ],
    )(s3, g3, dinv_w, batch_col, b3, Wl, bl)


def kernel(x, edge_index, batch, W1, b1, W2, b2, W3, b3, Wl, bl):
    src = edge_index[0].astype(jnp.int32)
    dst = edge_index[1].astype(jnp.int32)
    src0_rs = src.reshape(_RE, _CW)
    src1_rs = (src + _N).reshape(_RE, _CW)
    dst_rs = dst.reshape(_RE, _CW)
    batch_col = batch.astype(jnp.int32).reshape(_N, 1)
    zeros = jnp.zeros((_N, _HF), jnp.float32)

    dinv_w = _prep(dst_rs)
    g1 = _tc1(x, W1, dinv_w)
    s1 = _agg(g1.reshape(2 * _N, _HF), src0_rs, src1_rs, dst_rs, zeros)
    g2 = _tcmid(s1, g1, dinv_w, b1.reshape(1, _DH), W2)
    s2 = _agg(g2.reshape(2 * _N, _HF), src0_rs, src1_rs, dst_rs, zeros)
    g3 = _tcmid(s2, g2, dinv_w, b2.reshape(1, _DH), W3)
    s3 = _agg(g3.reshape(2 * _N, _HF), src0_rs, src1_rs, dst_rs, zeros)
    return _poolmm(s3, g3, dinv_w, batch_col, b3.reshape(1, _DH), Wl,
                   bl.reshape(1, 8))
